# baseline (speedup 1.0000x reference)
import functools

import jax
import jax.numpy as jnp
from jax import lax
from jax.experimental import pallas as pl
from jax.experimental.pallas import tpu as pltpu
from jax.experimental.pallas import tpu_sc as plsc

_NC = 2
_NS = 16
_NW = _NC * _NS


def _sc_gather(emb_pol, emb_tick, pol_ids, tick_ids, off, n):
    D = emb_pol.shape[1]
    bpw = n // _NW
    hw = bpw // 2
    mesh = plsc.VectorSubcoreMesh(core_axis_name="c", subcore_axis_name="s")

    @functools.partial(
        pl.kernel,
        mesh=mesh,
        out_type=jax.ShapeDtypeStruct((n, 2 * D), jnp.float32),
        scratch_types=[
            pltpu.VMEM((hw,), jnp.int32),
            pltpu.VMEM((hw,), jnp.int32),
            pltpu.VMEM((hw,), jnp.int32),
            pltpu.VMEM((hw,), jnp.int32),
            pltpu.VMEM((hw, D), jnp.float32),
            pltpu.VMEM((hw, D), jnp.float32),
            pltpu.VMEM((hw, D), jnp.float32),
            pltpu.VMEM((hw, D), jnp.float32),
            pltpu.SemaphoreType.DMA,
            pltpu.SemaphoreType.DMA,
            pltpu.SemaphoreType.DMA,
            pltpu.SemaphoreType.DMA,
        ],
    )
    def k(pol_hbm, tick_hbm, pid_hbm, tid_hbm, out_hbm,
          ip0, ip1, it0, it1, rp0, rp1, rt0, rt1, s0, s1, s2, s3):
        wid = lax.axis_index("s") * _NC + lax.axis_index("c")
        base = wid * bpw
        src = off + base
        lp0 = pltpu.async_copy(pid_hbm.at[pl.ds(src, hw)], ip0, s0)
        lp1 = pltpu.async_copy(pid_hbm.at[pl.ds(src + hw, hw)], ip1, s1)
        lt0 = pltpu.async_copy(tid_hbm.at[pl.ds(src, hw)], it0, s2)
        lt1 = pltpu.async_copy(tid_hbm.at[pl.ds(src + hw, hw)], it1, s3)
        lp0.wait()
        g0 = pltpu.async_copy(pol_hbm.at[ip0], rp0, s0)
        lp1.wait()
        g1 = pltpu.async_copy(pol_hbm.at[ip1], rp1, s1)
        lt0.wait()
        g2 = pltpu.async_copy(tick_hbm.at[it0], rt0, s2)
        lt1.wait()
        g3 = pltpu.async_copy(tick_hbm.at[it1], rt1, s3)
        g0.wait()
        w0 = pltpu.async_copy(
            rp0, out_hbm.at[pl.ds(base, hw), pl.ds(0, D)], s0)
        g1.wait()
        w1 = pltpu.async_copy(
            rp1, out_hbm.at[pl.ds(base + hw, hw), pl.ds(0, D)], s1)
        g2.wait()
        w2 = pltpu.async_copy(
            rt0, out_hbm.at[pl.ds(base, hw), pl.ds(D, D)], s2)
        g3.wait()
        w3 = pltpu.async_copy(
            rt1, out_hbm.at[pl.ds(base + hw, hw), pl.ds(D, D)], s3)
        w0.wait()
        w1.wait()
        w2.wait()
        w3.wait()

    return k(emb_pol, emb_tick, pol_ids, tick_ids)


def _tc_mlp(hcat, W1, W2, pack):
    B, K = hcat.shape
    H = W1.shape[1]
    H2 = W2.shape[1]
    BB = 2048

    def body(h_ref, w1, w2, pk, o_ref):
        hb = h_ref[...].astype(jnp.bfloat16)
        h = jnp.dot(hb, w1[...], preferred_element_type=jnp.float32)
        h = jnp.maximum(h + pk[0:1, :], 0.0).astype(jnp.bfloat16)
        h = jnp.dot(h, w2[...], preferred_element_type=jnp.float32)
        h = jnp.maximum(h + pk[1:2, :H2], 0.0)
        o = lax.dot_general(pk[1:2, H2:], h, (((1,), (1,)), ((), ())),
                            preferred_element_type=jnp.float32)
        o_ref[0] = o + pk[2:3, 0:1]

    out = pl.pallas_call(
        body,
        grid=(B // BB,),
        in_specs=[
            pl.BlockSpec((BB, K), lambda i: (i, 0)),
            pl.BlockSpec((K, H), lambda i: (0, 0)),
            pl.BlockSpec((H, H2), lambda i: (0, 0)),
            pl.BlockSpec((3, H), lambda i: (0, 0)),
        ],
        out_specs=pl.BlockSpec((1, 1, BB), lambda i: (i, 0, 0)),
        out_shape=jax.ShapeDtypeStruct((B // BB, 1, BB), jnp.float32),
    )(hcat, W1, W2, pack)
    return out.reshape(B)


def kernel(pol_ids, tick_ids, emb_pol, emb_tick, W1, b1, W2, b2, W3, b3):
    H = W1.shape[1]
    H2 = W2.shape[1]
    W1b = W1.astype(jnp.bfloat16)
    W2b = W2.astype(jnp.bfloat16)
    pack = jnp.zeros((3, H), jnp.float32)
    pack = pack.at[0].set(b1)
    pack = pack.at[1, :H2].set(b2)
    pack = pack.at[1, H2:].set(W3[:, 0])
    pack = pack.at[2, 0].set(b3[0])

    C = 2
    B = pol_ids.shape[0]
    CB = B // C
    pol_ids = pol_ids.astype(jnp.int32)
    tick_ids = tick_ids.astype(jnp.int32)
    hcats = [
        _sc_gather(emb_pol, emb_tick, pol_ids, tick_ids, c * CB, CB)
        for c in range(C)
    ]
    outs = [_tc_mlp(hc, W1b, W2b, pack) for hc in hcats]
    return jnp.concatenate(outs)


# speedup vs baseline: 1.0018x; 1.0018x over previous
import functools

import jax
import jax.numpy as jnp
from jax import lax
from jax.experimental import pallas as pl
from jax.experimental.pallas import tpu as pltpu
from jax.experimental.pallas import tpu_sc as plsc

_NC = 2
_NS = 16
_NW = _NC * _NS


def _sc_gather(emb_pol, emb_tick, pol_ids, tick_ids, off, n):
    D = emb_pol.shape[1]
    bpw = n // _NW
    hw = bpw // 2
    mesh = plsc.VectorSubcoreMesh(core_axis_name="c", subcore_axis_name="s")

    @functools.partial(
        pl.kernel,
        mesh=mesh,
        out_type=jax.ShapeDtypeStruct((n, 2 * D), jnp.float32),
        scratch_types=[
            pltpu.VMEM((hw,), jnp.int32),
            pltpu.VMEM((hw,), jnp.int32),
            pltpu.VMEM((hw,), jnp.int32),
            pltpu.VMEM((hw,), jnp.int32),
            pltpu.VMEM((hw, D), jnp.float32),
            pltpu.VMEM((hw, D), jnp.float32),
            pltpu.VMEM((hw, D), jnp.float32),
            pltpu.VMEM((hw, D), jnp.float32),
            pltpu.SemaphoreType.DMA,
            pltpu.SemaphoreType.DMA,
            pltpu.SemaphoreType.DMA,
            pltpu.SemaphoreType.DMA,
        ],
    )
    def k(pol_hbm, tick_hbm, pid_hbm, tid_hbm, out_hbm,
          ip0, ip1, it0, it1, rp0, rp1, rt0, rt1, s0, s1, s2, s3):
        wid = lax.axis_index("s") * _NC + lax.axis_index("c")
        base = wid * bpw
        src = off + base
        lp0 = pltpu.async_copy(pid_hbm.at[pl.ds(src, hw)], ip0, s0)
        lp1 = pltpu.async_copy(pid_hbm.at[pl.ds(src + hw, hw)], ip1, s1)
        lt0 = pltpu.async_copy(tid_hbm.at[pl.ds(src, hw)], it0, s2)
        lt1 = pltpu.async_copy(tid_hbm.at[pl.ds(src + hw, hw)], it1, s3)
        lp0.wait()
        g0 = pltpu.async_copy(pol_hbm.at[ip0], rp0, s0)
        lp1.wait()
        g1 = pltpu.async_copy(pol_hbm.at[ip1], rp1, s1)
        lt0.wait()
        g2 = pltpu.async_copy(tick_hbm.at[it0], rt0, s2)
        lt1.wait()
        g3 = pltpu.async_copy(tick_hbm.at[it1], rt1, s3)
        g0.wait()
        w0 = pltpu.async_copy(
            rp0, out_hbm.at[pl.ds(base, hw), pl.ds(0, D)], s0)
        g1.wait()
        w1 = pltpu.async_copy(
            rp1, out_hbm.at[pl.ds(base + hw, hw), pl.ds(0, D)], s1)
        g2.wait()
        w2 = pltpu.async_copy(
            rt0, out_hbm.at[pl.ds(base, hw), pl.ds(D, D)], s2)
        g3.wait()
        w3 = pltpu.async_copy(
            rt1, out_hbm.at[pl.ds(base + hw, hw), pl.ds(D, D)], s3)
        w0.wait()
        w1.wait()
        w2.wait()
        w3.wait()

    return k(emb_pol, emb_tick, pol_ids, tick_ids)


def _tc_mlp(hcat, W1, W2, pack):
    B, K = hcat.shape
    H = W1.shape[1]
    H2 = W2.shape[1]
    BB = 2048

    def body(h_ref, w1, w2, pk, o_ref):
        hb = h_ref[...].astype(jnp.bfloat16)
        h = jnp.dot(hb, w1[...], preferred_element_type=jnp.float32)
        h = jnp.maximum(h + pk[0:1, :], 0.0).astype(jnp.bfloat16)
        h = jnp.dot(h, w2[...], preferred_element_type=jnp.float32)
        h = jnp.maximum(h + pk[1:2, :H2], 0.0)
        o = lax.dot_general(pk[1:2, H2:], h, (((1,), (1,)), ((), ())),
                            preferred_element_type=jnp.float32)
        o_ref[...] = (o + pk[2:3, 0:1]).reshape(o.shape[1])

    out = pl.pallas_call(
        body,
        grid=(B // BB,),
        in_specs=[
            pl.BlockSpec((BB, K), lambda i: (i, 0)),
            pl.BlockSpec((K, H), lambda i: (0, 0)),
            pl.BlockSpec((H, H2), lambda i: (0, 0)),
            pl.BlockSpec((3, H), lambda i: (0, 0)),
        ],
        out_specs=pl.BlockSpec((BB,), lambda i: (i,)),
        out_shape=jax.ShapeDtypeStruct((B,), jnp.float32),
    )(hcat, W1, W2, pack)
    return out


def kernel(pol_ids, tick_ids, emb_pol, emb_tick, W1, b1, W2, b2, W3, b3):
    H = W1.shape[1]
    H2 = W2.shape[1]
    W1b = W1.astype(jnp.bfloat16)
    W2b = W2.astype(jnp.bfloat16)
    pack = jnp.zeros((3, H), jnp.float32)
    pack = pack.at[0].set(b1)
    pack = pack.at[1, :H2].set(b2)
    pack = pack.at[1, H2:].set(W3[:, 0])
    pack = pack.at[2, 0].set(b3[0])

    C = 2
    B = pol_ids.shape[0]
    CB = B // C
    pol_ids = pol_ids.astype(jnp.int32)
    tick_ids = tick_ids.astype(jnp.int32)
    hcats = [
        _sc_gather(emb_pol, emb_tick, pol_ids, tick_ids, c * CB, CB)
        for c in range(C)
    ]
    outs = [_tc_mlp(hc, W1b, W2b, pack) for hc in hcats]
    return jnp.concatenate(outs)
